# Initial kernel scaffold; baseline (speedup 1.0000x reference)
import jax
import jax.numpy as jnp
from jax.experimental import pallas as pl


def kernel(tokens, tables, level_scale):
    raise NotImplementedError("write your pallas kernel here")


# speedup vs baseline: 6.3042x; 6.3042x over previous
import functools

import jax
import jax.numpy as jnp
from jax import lax
from jax.experimental import pallas as pl
from jax.experimental.pallas import tpu as pltpu
from jax.experimental.pallas import tpu_sc as plsc

C = 8
V = 2048
D = 64
LANES = 16

_info = plsc.get_sparse_core_info()
_NC, _NS = _info.num_cores, _info.num_subcores
NW = _NC * _NS


@functools.lru_cache(maxsize=None)
def _build(rows):
    rows_per_w = rows // NW
    chunk = 16
    nchunk = rows_per_w // chunk
    mesh = plsc.VectorSubcoreMesh(core_axis_name="c", subcore_axis_name="s")

    @functools.partial(
        pl.kernel,
        mesh=mesh,
        out_type=jax.ShapeDtypeStruct((rows, D), jnp.float32),
        compiler_params=pltpu.CompilerParams(use_tc_tiling_on_sc=False),
        scratch_types=[
            pltpu.VMEM((rows_per_w * C,), jnp.int32),
            pltpu.VMEM((nchunk, chunk * C), jnp.int32),
            pltpu.VMEM((chunk * C, D), jnp.float32),
            pltpu.VMEM((chunk, D), jnp.float32),
            pltpu.VMEM((C, LANES), jnp.float32),
            pltpu.SemaphoreType.DMA,
        ],
    )
    def k(tok_hbm, table_hbm, scale_hbm, out_hbm,
          tok_v, idx_v, buf_v, outb_v, scale_v, sem):
        wid = lax.axis_index("s") * _NC + lax.axis_index("c")
        base = wid * rows_per_w
        pltpu.sync_copy(scale_hbm, scale_v)
        pltpu.sync_copy(tok_hbm.at[pl.ds(base * C, rows_per_w * C)], tok_v)

        offs = (lax.iota(jnp.int32, LANES) & 7) * V
        svec = [scale_v[c, :] for c in range(C)]

        def idx_body(j, carry):
            for t in range(chunk * C // LANES):
                tv = tok_v[pl.ds(j * chunk * C + t * LANES, LANES)]
                idx_v[j, pl.ds(t * LANES, LANES)] = tv + offs
            return carry

        lax.fori_loop(0, nchunk, idx_body, 0)

        def body(j, carry):
            pltpu.async_copy(table_hbm.at[idx_v.at[j]], buf_v, sem).wait()
            for i in range(chunk):
                for g in range(D // LANES):
                    acc = buf_v[i * C, pl.ds(g * LANES, LANES)] * svec[0]
                    for c in range(1, C):
                        acc = acc + buf_v[i * C + c, pl.ds(g * LANES, LANES)] * svec[c]
                    outb_v[i, pl.ds(g * LANES, LANES)] = acc
            pltpu.sync_copy(outb_v, out_hbm.at[pl.ds(base + j * chunk, chunk)])
            return carry

        lax.fori_loop(0, nchunk, body, 0)

    return k


def kernel(tokens, tables, level_scale):
    b, l, _ = tokens.shape
    rows = b * l
    tok = tokens.astype(jnp.int32).reshape(rows * C)
    table = tables.reshape(C * V, D)
    scale_b = jnp.broadcast_to(level_scale.reshape(C, 1), (C, LANES))
    out = _build(rows)(tok, table, scale_b)
    return out.reshape(b, l, D)


# speedup vs baseline: 7.5884x; 1.2037x over previous
import functools

import jax
import jax.numpy as jnp
from jax import lax
from jax.experimental import pallas as pl
from jax.experimental.pallas import tpu as pltpu
from jax.experimental.pallas import tpu_sc as plsc

C = 8
V = 2048
D = 64
LANES = 16

_info = plsc.get_sparse_core_info()
_NC, _NS = _info.num_cores, _info.num_subcores
NW = _NC * _NS


NBUF = 4


@functools.lru_cache(maxsize=None)
def _build(rows):
    rows_per_w = rows // NW
    chunk = 16
    nchunk = rows_per_w // chunk
    nouter = nchunk // NBUF
    mesh = plsc.VectorSubcoreMesh(core_axis_name="c", subcore_axis_name="s")

    @functools.partial(
        pl.kernel,
        mesh=mesh,
        out_type=jax.ShapeDtypeStruct((rows, D), jnp.float32),
        compiler_params=pltpu.CompilerParams(use_tc_tiling_on_sc=False),
        scratch_types=[
            pltpu.VMEM((rows_per_w * C,), jnp.int32),
            pltpu.VMEM((nchunk, chunk * C), jnp.int32),
            pltpu.VMEM((NBUF, chunk * C, D), jnp.float32),
            pltpu.VMEM((NBUF, chunk, D), jnp.float32),
            pltpu.VMEM((C, LANES), jnp.float32),
            [pltpu.SemaphoreType.DMA] * NBUF,
            [pltpu.SemaphoreType.DMA] * NBUF,
        ],
    )
    def k(tok_hbm, table_hbm, scale_hbm, out_hbm,
          tok_v, idx_v, buf_v, outb_v, scale_v, gsems, ssems):
        wid = lax.axis_index("s") * _NC + lax.axis_index("c")
        base = wid * rows_per_w
        pltpu.sync_copy(scale_hbm, scale_v)
        pltpu.sync_copy(tok_hbm.at[pl.ds(base * C, rows_per_w * C)], tok_v)

        offs = (lax.iota(jnp.int32, LANES) & 7) * V
        svec = [scale_v[c, :] for c in range(C)]

        def idx_body(j, carry):
            for t in range(chunk * C // LANES):
                tv = tok_v[pl.ds(j * chunk * C + t * LANES, LANES)]
                idx_v[j, pl.ds(t * LANES, LANES)] = tv + offs
            return carry

        lax.fori_loop(0, nchunk, idx_body, 0)

        def gather(j, b):
            pltpu.async_copy(table_hbm.at[idx_v.at[j]], buf_v.at[b], gsems[b])

        for b in range(NBUF):
            gather(b, b)

        def body(m, carry):
            for b in range(NBUF):
                j = m * NBUF + b
                pltpu.make_async_copy(table_hbm.at[idx_v.at[j]],
                                      buf_v.at[b], gsems[b]).wait()

                @pl.when(m > 0)
                def _():
                    pltpu.make_async_copy(
                        outb_v.at[b],
                        out_hbm.at[pl.ds(base + (j - NBUF) * chunk, chunk)],
                        ssems[b]).wait()

                for i in range(chunk):
                    for g in range(D // LANES):
                        acc = buf_v[b, i * C, pl.ds(g * LANES, LANES)] * svec[0]
                        for c in range(1, C):
                            acc = acc + buf_v[b, i * C + c,
                                              pl.ds(g * LANES, LANES)] * svec[c]
                        outb_v[b, i, pl.ds(g * LANES, LANES)] = acc
                pltpu.async_copy(outb_v.at[b],
                                 out_hbm.at[pl.ds(base + j * chunk, chunk)],
                                 ssems[b])

                @pl.when(m < nouter - 1)
                def _():
                    gather(j + NBUF, b)
            return carry

        lax.fori_loop(0, nouter, body, 0)

        for b in range(NBUF):
            j = (nouter - 1) * NBUF + b
            pltpu.make_async_copy(outb_v.at[b],
                                  out_hbm.at[pl.ds(base + j * chunk, chunk)],
                                  ssems[b]).wait()

    return k


def kernel(tokens, tables, level_scale):
    b, l, _ = tokens.shape
    rows = b * l
    tok = tokens.astype(jnp.int32).reshape(rows * C)
    table = tables.reshape(C * V, D)
    scale_b = jnp.broadcast_to(level_scale.reshape(C, 1), (C, LANES))
    out = _build(rows)(tok, table, scale_b)
    return out.reshape(b, l, D)


# speedup vs baseline: 7.9722x; 1.0506x over previous
import functools

import jax
import jax.numpy as jnp
from jax import lax
from jax.experimental import pallas as pl
from jax.experimental.pallas import tpu as pltpu
from jax.experimental.pallas import tpu_sc as plsc

C = 8
V = 2048
D = 64
LANES = 16

_info = plsc.get_sparse_core_info()
_NC, _NS = _info.num_cores, _info.num_subcores
NW = _NC * _NS


NBUF = 2


@functools.lru_cache(maxsize=None)
def _build(rows):
    rows_per_w = rows // NW
    chunk = 16
    nchunk = rows_per_w // chunk
    nouter = nchunk // NBUF
    mesh = plsc.VectorSubcoreMesh(core_axis_name="c", subcore_axis_name="s")

    @functools.partial(
        pl.kernel,
        mesh=mesh,
        out_type=jax.ShapeDtypeStruct((rows, D), jnp.float32),
        compiler_params=pltpu.CompilerParams(use_tc_tiling_on_sc=False),
        scratch_types=[
            pltpu.VMEM((nchunk, chunk * C), jnp.int32),
            pltpu.VMEM((NBUF, chunk * C, D), jnp.float32),
            pltpu.VMEM((NBUF, chunk, D), jnp.float32),
            pltpu.VMEM((C, LANES), jnp.float32),
            pltpu.VMEM_SHARED((C * V, D), jnp.float32),
            [pltpu.SemaphoreType.DMA] * NBUF,
            [pltpu.SemaphoreType.DMA] * NBUF,
        ],
    )
    def k(tok_hbm, table_hbm, scale_hbm, out_hbm,
          idx_v, buf_v, outb_v, scale_v, spt, gsems, ssems):
        wid = lax.axis_index("s") * _NC + lax.axis_index("c")
        base = wid * rows_per_w
        sid = lax.axis_index("s")
        tchunk = C * V // _NS
        tstep = chunk * C
        def fill_body(p, carry):
            r = sid * tchunk + p * tstep
            pltpu.sync_copy(table_hbm.at[pl.ds(r, tstep)], buf_v.at[0])
            pltpu.sync_copy(buf_v.at[0], spt.at[pl.ds(r, tstep)])
            return carry

        lax.fori_loop(0, tchunk // tstep, fill_body, 0)
        pltpu.sync_copy(scale_hbm, scale_v)
        pltpu.sync_copy(tok_hbm.at[pl.ds(wid * nchunk, nchunk)], idx_v)

        offs = (lax.iota(jnp.int32, LANES) & 7) * V
        svec = [scale_v[c, :] for c in range(C)]

        def idx_body(j, carry):
            for t in range(chunk * C // LANES):
                sl = pl.ds(t * LANES, LANES)
                idx_v[j, sl] = idx_v[j, sl] + offs
            return carry

        lax.fori_loop(0, nchunk, idx_body, 0)
        plsc.subcore_barrier()

        def gather(j, b):
            pltpu.async_copy(spt.at[idx_v.at[j]], buf_v.at[b], gsems[b])

        for b in range(NBUF):
            gather(b, b)

        def body(m, carry):
            for b in range(NBUF):
                j = m * NBUF + b
                pltpu.make_async_copy(spt.at[idx_v.at[j]],
                                      buf_v.at[b], gsems[b]).wait()

                @pl.when(m > 0)
                def _():
                    pltpu.make_async_copy(
                        outb_v.at[b],
                        out_hbm.at[pl.ds(base + (j - NBUF) * chunk, chunk)],
                        ssems[b]).wait()

                for i in range(chunk):
                    for g in range(D // LANES):
                        acc = buf_v[b, i * C, pl.ds(g * LANES, LANES)] * svec[0]
                        for c in range(1, C):
                            acc = acc + buf_v[b, i * C + c,
                                              pl.ds(g * LANES, LANES)] * svec[c]
                        outb_v[b, i, pl.ds(g * LANES, LANES)] = acc
                pltpu.async_copy(outb_v.at[b],
                                 out_hbm.at[pl.ds(base + j * chunk, chunk)],
                                 ssems[b])

                @pl.when(m < nouter - 1)
                def _():
                    gather(j + NBUF, b)
            return carry

        lax.fori_loop(0, nouter, body, 0)

        for b in range(NBUF):
            j = (nouter - 1) * NBUF + b
            pltpu.make_async_copy(outb_v.at[b],
                                  out_hbm.at[pl.ds(base + j * chunk, chunk)],
                                  ssems[b]).wait()

    return k


def kernel(tokens, tables, level_scale):
    b, l, _ = tokens.shape
    rows = b * l
    tok = tokens.astype(jnp.int32).reshape(rows * C // 128, 128)
    table = tables.reshape(C * V, D)
    scale_b = jnp.broadcast_to(level_scale.reshape(C, 1), (C, LANES))
    out = _build(rows)(tok, table, scale_b)
    return out.reshape(b, l, D)
